# baseline (speedup 1.0000x reference)
import numpy as np

import jax
import jax.numpy as jnp
from jax.experimental import pallas as pl
from jax.experimental.pallas import tpu as pltpu

_C = 384
_HW = 784
_W = 28
_B = 64
_RED = 24
_E = 16
_PAD = 16


def _conv_pool_kernel(xa_ref, xb_ref, xc_ref, xd_ref, ident_ref, w9_ref,
                      scale_ref, bias_ref, pw_ref,
                      out_ref, xt0_ref, xt1_ref, xt2_ref, act_ref):
    xts = (xt0_ref, xt1_ref, xt2_ref)
    x = jnp.concatenate(
        [xa_ref[0, 0], xb_ref[0, 0], xc_ref[0, 0], xd_ref[0, 0]],
        axis=0)
    xt = jnp.transpose(x)
    zpad16 = jnp.zeros((_PAD, 128), jnp.float32)
    for k in range(3):
        xts[k][0:_PAD, :] = zpad16
        xts[k][pl.ds(_PAD, _HW), :] = xt[:, 128 * k:128 * (k + 1)]
        xts[k][pl.ds(_PAD + _HW, _PAD), :] = zpad16

    rid = jax.lax.broadcasted_iota(jnp.int32, (14, 128), 0)
    m0 = (rid != 0).astype(jnp.float32)
    zpad2 = jnp.zeros((2, _C), jnp.float32)

    for i in range(14):
        for k in range(3):
            g = [None, None, None]
            for dj in range(3):
                for di in range(3):
                    if i == 0 and di == 0:
                        continue
                    start = 56 * i + 28 * di + dj - 29 + _PAD
                    t = xts[k][pl.ds(start, 14, 2), :]
                    t = t * w9_ref[3 * di + dj:3 * di + dj + 1,
                                   128 * k:128 * (k + 1)]
                    g[dj] = t if g[dj] is None else g[dj] + t
            acc = g[0] * m0 + g[1] + g[2]
            a = acc * scale_ref[0:1, 128 * k:128 * (k + 1)] \
                + bias_ref[0:1, 128 * k:128 * (k + 1)]
            a = a * jax.nn.sigmoid(a)
            act_ref[i, 0:14, 128 * k:128 * (k + 1)] = a
        act_ref[i, 14:16, :] = zpad2

    act = act_ref[...].reshape(14 * 16, _C)
    z = jnp.dot(act, pw_ref[...], preferred_element_type=jnp.float32)
    z = z * jax.nn.sigmoid(z)
    pooled = jnp.sum(z, axis=0) * (1.0 / 196.0)
    out_ref[0, 0, :] = pooled


def _routing_kernel(p_ref, fcT_ref, fcb_ref, w_ref, i_ref, l_ref):
    logits = (jnp.dot(p_ref[...], fcT_ref[...],
                      preferred_element_type=jnp.float32) + fcb_ref[...])
    lanes = jax.lax.broadcasted_iota(jnp.int32, (_B, _E), 1)
    m1 = jnp.max(logits, axis=1, keepdims=True)
    idx1 = jnp.min(jnp.where(logits == m1, lanes, _E), axis=1, keepdims=True)
    l2 = jnp.where(lanes == idx1, -1e30, logits)
    m2 = jnp.max(l2, axis=1, keepdims=True)
    idx2 = jnp.min(jnp.where(l2 == m2, lanes, _E), axis=1, keepdims=True)
    e = jnp.exp(logits - m1)
    probs = e / jnp.sum(e, axis=1, keepdims=True)
    s1 = jnp.sum(jnp.where(lanes == idx1, probs, 0.0), axis=1, keepdims=True)
    s2 = jnp.sum(jnp.where(lanes == idx2, probs, 0.0), axis=1, keepdims=True)
    inv = 1.0 / (s1 + s2 + 1e-6)
    w_ref[...] = jnp.concatenate([s1 * inv, s2 * inv], axis=1)
    i_ref[...] = jnp.concatenate([idx1, idx2], axis=1)
    l_ref[...] = logits


def kernel(x, dw_w, bn_gamma, bn_beta, bn_mean, bn_var, pw_w, fc_w, fc_b):
    xf = x.reshape(_B, 4, _C // 4, _HW)
    w9 = dw_w.reshape(_C, 9).T
    inv_std = jax.lax.rsqrt(bn_var + 1e-5)
    scale = (bn_gamma * inv_std)[None, :]
    bias = (bn_beta - bn_mean * bn_gamma * inv_std)[None, :]
    pwT = pw_w.reshape(_RED, _C).T
    ident = jnp.eye(_C, dtype=jnp.float32)

    pooled3 = pl.pallas_call(
        _conv_pool_kernel,
        grid=(_B,),
        in_specs=[
            pl.BlockSpec((1, 1, _C // 4, _HW), lambda b: (b, 0, 0, 0)),
            pl.BlockSpec((1, 1, _C // 4, _HW), lambda b: (b, 1, 0, 0)),
            pl.BlockSpec((1, 1, _C // 4, _HW), lambda b: (b, 2, 0, 0)),
            pl.BlockSpec((1, 1, _C // 4, _HW), lambda b: (b, 3, 0, 0)),
            pl.BlockSpec((_C, _C), lambda b: (0, 0)),
            pl.BlockSpec((9, _C), lambda b: (0, 0)),
            pl.BlockSpec((1, _C), lambda b: (0, 0)),
            pl.BlockSpec((1, _C), lambda b: (0, 0)),
            pl.BlockSpec((_C, _RED), lambda b: (0, 0)),
        ],
        out_specs=pl.BlockSpec((1, 1, _RED), lambda b: (b, 0, 0)),
        out_shape=jax.ShapeDtypeStruct((_B, 1, _RED), jnp.float32),
        scratch_shapes=[
            pltpu.VMEM((_HW + 2 * _PAD, 128), jnp.float32),
            pltpu.VMEM((_HW + 2 * _PAD, 128), jnp.float32),
            pltpu.VMEM((_HW + 2 * _PAD, 128), jnp.float32),
            pltpu.VMEM((14, 16, _C), jnp.float32),
        ],
    )(xf, xf, xf, xf, ident, w9, scale, bias, pwT)

    pooled = pooled3.reshape(_B, _RED)
    weights, idx, logits = pl.pallas_call(
        _routing_kernel,
        out_shape=[
            jax.ShapeDtypeStruct((_B, 2), jnp.float32),
            jax.ShapeDtypeStruct((_B, 2), jnp.int32),
            jax.ShapeDtypeStruct((_B, _E), jnp.float32),
        ],
    )(pooled, fc_w.T, fc_b[None, :])
    return (weights, idx, logits)


# speedup vs baseline: 2.3720x; 2.3720x over previous
import numpy as np

import jax
import jax.numpy as jnp
from jax.experimental import pallas as pl
from jax.experimental.pallas import tpu as pltpu

_C = 384
_HW = 784
_W = 28
_B = 64
_RED = 24
_E = 16
_PAD = 16


def _conv_pool_kernel(x_ref, ident_ref, w9_ref, scale_ref, bias_ref, pw_ref,
                      out_ref, xt0_ref, xt1_ref, xt2_ref, act_ref):
    xts = (xt0_ref, xt1_ref, xt2_ref)
    for bb in range(4):
        _one_batch(bb, x_ref, w9_ref, scale_ref, bias_ref, pw_ref, out_ref,
                   xts, act_ref)


def _one_batch(bb, x_ref, w9_ref, scale_ref, bias_ref, pw_ref, out_ref,
               xts, act_ref):
    x = x_ref[bb]
    xt = jnp.transpose(x)
    zpad16 = jnp.zeros((_PAD, 128), jnp.float32)
    for k in range(3):
        xts[k][0:_PAD, :] = zpad16
        xts[k][pl.ds(_PAD, _HW), :] = xt[:, 128 * k:128 * (k + 1)]
        xts[k][pl.ds(_PAD + _HW, _PAD), :] = zpad16

    rid = jax.lax.broadcasted_iota(jnp.int32, (14, 128), 0)
    m0 = (rid != 0).astype(jnp.float32)
    zpad2 = jnp.zeros((2, _C), jnp.float32)

    for i in range(14):
        for k in range(3):
            g = [None, None, None]
            for dj in range(3):
                for di in range(3):
                    if i == 0 and di == 0:
                        continue
                    start = 56 * i + 28 * di + dj - 29 + _PAD
                    t = xts[k][pl.ds(start, 14, 2), :]
                    t = t * w9_ref[3 * di + dj:3 * di + dj + 1,
                                   128 * k:128 * (k + 1)]
                    g[dj] = t if g[dj] is None else g[dj] + t
            acc = g[0] * m0 + g[1] + g[2]
            a = acc * scale_ref[0:1, 128 * k:128 * (k + 1)] \
                + bias_ref[0:1, 128 * k:128 * (k + 1)]
            a = a * jax.nn.sigmoid(a)
            act_ref[i, 0:14, 128 * k:128 * (k + 1)] = a
        act_ref[i, 14:16, :] = zpad2

    act = act_ref[...].reshape(14 * 16, _C)
    z = jnp.dot(act, pw_ref[...], preferred_element_type=jnp.float32)
    z = z * jax.nn.sigmoid(z)
    pooled = jnp.sum(z, axis=0) * (1.0 / 196.0)
    out_ref[bb, 0, :] = pooled


def _routing_kernel(p_ref, fcT_ref, fcb_ref, w_ref, i_ref, l_ref):
    logits = (jnp.dot(p_ref[...], fcT_ref[...],
                      preferred_element_type=jnp.float32) + fcb_ref[...])
    lanes = jax.lax.broadcasted_iota(jnp.int32, (_B, _E), 1)
    m1 = jnp.max(logits, axis=1, keepdims=True)
    idx1 = jnp.min(jnp.where(logits == m1, lanes, _E), axis=1, keepdims=True)
    l2 = jnp.where(lanes == idx1, -1e30, logits)
    m2 = jnp.max(l2, axis=1, keepdims=True)
    idx2 = jnp.min(jnp.where(l2 == m2, lanes, _E), axis=1, keepdims=True)
    e = jnp.exp(logits - m1)
    probs = e / jnp.sum(e, axis=1, keepdims=True)
    s1 = jnp.sum(jnp.where(lanes == idx1, probs, 0.0), axis=1, keepdims=True)
    s2 = jnp.sum(jnp.where(lanes == idx2, probs, 0.0), axis=1, keepdims=True)
    inv = 1.0 / (s1 + s2 + 1e-6)
    w_ref[...] = jnp.concatenate([s1 * inv, s2 * inv], axis=1)
    i_ref[...] = jnp.concatenate([idx1, idx2], axis=1)
    l_ref[...] = logits


def kernel(x, dw_w, bn_gamma, bn_beta, bn_mean, bn_var, pw_w, fc_w, fc_b):
    xf = x.reshape(_B, _C, _HW)
    w9 = dw_w.reshape(_C, 9).T
    inv_std = jax.lax.rsqrt(bn_var + 1e-5)
    scale = (bn_gamma * inv_std)[None, :]
    bias = (bn_beta - bn_mean * bn_gamma * inv_std)[None, :]
    pwT = pw_w.reshape(_RED, _C).T
    ident = jnp.eye(_C, dtype=jnp.float32)

    pooled3 = pl.pallas_call(
        _conv_pool_kernel,
        grid=(_B // 4,),
        in_specs=[
            pl.BlockSpec((4, _C, _HW), lambda b: (b, 0, 0)),
            pl.BlockSpec((_C, _C), lambda b: (0, 0)),
            pl.BlockSpec((9, _C), lambda b: (0, 0)),
            pl.BlockSpec((1, _C), lambda b: (0, 0)),
            pl.BlockSpec((1, _C), lambda b: (0, 0)),
            pl.BlockSpec((_C, _RED), lambda b: (0, 0)),
        ],
        out_specs=pl.BlockSpec((4, 1, _RED), lambda b: (b, 0, 0)),
        out_shape=jax.ShapeDtypeStruct((_B, 1, _RED), jnp.float32),
        scratch_shapes=[
            pltpu.VMEM((_HW + 2 * _PAD, 128), jnp.float32),
            pltpu.VMEM((_HW + 2 * _PAD, 128), jnp.float32),
            pltpu.VMEM((_HW + 2 * _PAD, 128), jnp.float32),
            pltpu.VMEM((14, 16, _C), jnp.float32),
        ],
    )(xf, ident, w9, scale, bias, pwT)

    pooled = pooled3.reshape(_B, _RED)
    weights, idx, logits = pl.pallas_call(
        _routing_kernel,
        out_shape=[
            jax.ShapeDtypeStruct((_B, 2), jnp.float32),
            jax.ShapeDtypeStruct((_B, 2), jnp.int32),
            jax.ShapeDtypeStruct((_B, _E), jnp.float32),
        ],
    )(pooled, fc_w.T, fc_b[None, :])
    return (weights, idx, logits)
